# Initial kernel scaffold; baseline (speedup 1.0000x reference)
import jax
import jax.numpy as jnp
from jax.experimental import pallas as pl


def kernel(inputs, table, W1, b1, W2, b2):
    raise NotImplementedError("write your pallas kernel here")


# speedup vs baseline: 3.1976x; 3.1976x over previous
import functools

import jax
import jax.numpy as jnp
from jax import lax
from jax.experimental import pallas as pl
from jax.experimental.pallas import tpu as pltpu
from jax.experimental.pallas import tpu_sc as plsc

B = 16384
H = 200
E = 64
HID = 256

NW = 32
BPW = B // NW
CB = 4
NIDX = CB * H
NCH = BPW // CB
UNROLL = 8


def _sc_pool(idx_hbm, table_hbm, out_hbm,
             idx0, idx1, rows0, rows1, stage, sem0, sem1):
    wid = lax.axis_index("s") * 2 + lax.axis_index("c")
    base_row = wid * BPW

    idxs = (idx0, idx1)
    rows = (rows0, rows1)
    sems = (sem0, sem1)

    def fetch(chunk, b):
        start = pl.multiple_of((base_row + chunk * CB) * H, NIDX)
        pltpu.sync_copy(idx_hbm.at[pl.ds(start, NIDX)], idxs[b])
        pltpu.async_copy(table_hbm.at[idxs[b]], rows[b], sems[b])

    fetch(0, 0)
    fetch(1, 1)

    def outer(g, _):
        for b in range(2):
            chunk = g * 2 + b
            pltpu.make_async_copy(table_hbm.at[idxs[b]], rows[b],
                                  sems[b]).wait()
            for r in range(CB):
                def jbody(jj, accs, r=r, b=b):
                    accs = list(accs)
                    for u in range(UNROLL):
                        row = (jj * UNROLL + u) + r * H
                        for c in range(E // 16):
                            accs[c] = accs[c] + rows[b][row, pl.ds(c * 16, 16)]
                    return tuple(accs)

                zero = jnp.zeros((16,), jnp.float32)
                accs = lax.fori_loop(0, H // UNROLL, jbody,
                                     (zero,) * (E // 16))
                for c in range(E // 16):
                    stage[r, pl.ds(c * 16, 16)] = accs[c]
            out_start = pl.multiple_of(base_row + chunk * CB, CB)
            pltpu.sync_copy(stage, out_hbm.at[pl.ds(out_start, CB)])
            @pl.when(chunk + 2 < NCH)
            def _(b=b, chunk=chunk):
                fetch(chunk + 2, b)
        return _

    lax.fori_loop(0, NCH // 2, outer, None)


def _mlp_body(x_ref, w1_ref, b1_ref, w2_ref, b2_ref, o_ref):
    x = x_ref[...] * (1.0 / H)
    h = jnp.dot(x, w1_ref[...], preferred_element_type=jnp.float32)
    h = jnp.maximum(h + b1_ref[...], 0.0)
    z = jnp.sum(h * w2_ref[...], axis=1, keepdims=True) + b2_ref[...]
    o_ref[...] = 1.0 / (1.0 + jnp.exp(-z))


def kernel(inputs, table, W1, b1, W2, b2):
    idx_flat = inputs.reshape(-1).astype(jnp.int32)

    mesh = plsc.VectorSubcoreMesh(core_axis_name="c", subcore_axis_name="s")
    pooled = pl.kernel(
        _sc_pool,
        out_type=jax.ShapeDtypeStruct((B, E), jnp.float32),
        mesh=mesh,
        compiler_params=pltpu.CompilerParams(use_tc_tiling_on_sc=False),
        scratch_types=[
            pltpu.VMEM((NIDX,), jnp.int32),
            pltpu.VMEM((NIDX,), jnp.int32),
            pltpu.VMEM((NIDX, E), jnp.float32),
            pltpu.VMEM((NIDX, E), jnp.float32),
            pltpu.VMEM((CB, E), jnp.float32),
            pltpu.SemaphoreType.DMA,
            pltpu.SemaphoreType.DMA,
        ],
    )(idx_flat, table)

    BM = 2048
    out = pl.pallas_call(
        _mlp_body,
        grid=(B // BM,),
        in_specs=[
            pl.BlockSpec((BM, E), lambda i: (i, 0)),
            pl.BlockSpec((E, HID), lambda i: (0, 0)),
            pl.BlockSpec((1, HID), lambda i: (0, 0)),
            pl.BlockSpec((1, HID), lambda i: (0, 0)),
            pl.BlockSpec((1, 1), lambda i: (0, 0)),
        ],
        out_specs=pl.BlockSpec((BM, 1), lambda i: (i, 0)),
        out_shape=jax.ShapeDtypeStruct((B, 1), jnp.float32),
    )(pooled, W1, b1.reshape(1, HID), W2.reshape(1, HID), b2.reshape(1, 1))
    return out
